# baseline (speedup 1.0000x reference)
import functools

import jax
import jax.numpy as jnp
from jax import lax
from jax.experimental import pallas as pl
from jax.experimental.pallas import tpu as pltpu
from jax.experimental.pallas import tpu_sc as plsc

N, E, D, ED = 10000, 160000, 256, 16
K1, K2, K3 = 2000, 400, 80
BE = 64
NB = E // BE
NW = 32
TB = (NB + NW - 1) // NW
E_PAD = 163840
NBP = E_PAD // BE
DUMP = 2048
F = 64


def _le_body(ea_ref, lew_ref, leb_ref, out_ref):
    out_ref[...] = (
        jnp.dot(ea_ref[...], lew_ref[...], preferred_element_type=jnp.float32)
        + leb_ref[...]
    )


def _le_call(ea, lew, leb, P):
    e_pad = ea.shape[0]
    lewp = lew.reshape(ED, P, F).transpose(1, 0, 2)
    lebp = leb.reshape(P, 1, F)
    return pl.pallas_call(
        _le_body,
        grid=(e_pad // 640, P),
        in_specs=[
            pl.BlockSpec((640, ED), lambda i, p: (i, 0)),
            pl.BlockSpec((None, ED, F), lambda i, p: (p, 0, 0)),
            pl.BlockSpec((None, 1, F), lambda i, p: (p, 0, 0)),
        ],
        out_specs=pl.BlockSpec((None, 640, F), lambda i, p: (p, i, 0)),
        out_shape=jax.ShapeDtypeStruct((P, e_pad, F), jnp.float32),
    )(ea, lewp, lebp)


def _le23_body(ea_ref, w_ref, b_ref, out_ref):
    out_ref[...] = (
        jnp.dot(ea_ref[...], w_ref[...], preferred_element_type=jnp.float32)
        + b_ref[...]
    )


def _le23_call(ea, lew2, leb2, lew3, leb3):
    e_pad = ea.shape[0]
    w23 = jnp.concatenate([lew2, lew3], axis=1)
    b23 = jnp.concatenate([leb2, leb3]).reshape(1, 128)
    return pl.pallas_call(
        _le23_body,
        grid=(e_pad // 640,),
        in_specs=[
            pl.BlockSpec((640, ED), lambda i: (i, 0)),
            pl.BlockSpec((ED, 128), lambda i: (0, 0)),
            pl.BlockSpec((1, 128), lambda i: (0, 0)),
        ],
        out_specs=pl.BlockSpec((640, 128), lambda i: (i, 0)),
        out_shape=jax.ShapeDtypeStruct((e_pad, 128), jnp.float32),
    )(ea, w23, b23)


def _agg_body(P, n_acc, NT, coffs, nb, legather, cole, *refs):
    xts = refs[0:NT]
    nles = 1 if legather else P
    les = refs[NT:NT + nles]
    src2d, dst2d = refs[NT + nles], refs[NT + nles + 1]
    i = NT + nles + 2
    if legather:
        eid2d = refs[i]
        i += 1
    outs = refs[i:i + P]
    (acc, src_v0, src_v1, dst_v0, dst_v1, eid_v0, eid_v1, rows_v0, rows_v1,
     le_v0, le_v1, upd_v, si0, si1, dl0, dl1, gs0, gs1, gl0, gl1,
     ss) = refs[i + P:]
    src_v = (src_v0, src_v1)
    dst_v = (dst_v0, dst_v1)
    eid_v = (eid_v0, eid_v1)
    rows_v = (rows_v0, rows_v1)
    le_v = (le_v0, le_v1)
    si = (si0, si1)
    dl = (dl0, dl1)
    gs = (gs0, gs1)
    gl = (gl0, gl1)
    T = nb // NW

    cid = lax.axis_index("c")
    sid = lax.axis_index("s")
    wid = sid * 2 + cid
    rpt = n_acc // 16

    for p in range(P):
        xt = xts[p // (P // NT)]
        co = coffs[p]

        def zero_upd(r, _):
            for j in range(8):
                upd_v[r, pl.ds(16 * j, 16)] = jnp.zeros((16,), jnp.float32)
            return 0

        lax.fori_loop(0, BE, zero_upd, 0)
        for t in range(rpt // BE):
            pltpu.sync_copy(upd_v, acc.at[pl.ds(sid * rpt + t * BE, BE)])
        plsc.subcore_barrier()

        def issue_idx(t, s):
            b = wid + NW * t
            pltpu.async_copy(src2d.at[b], src_v[s], si[s])
            pltpu.async_copy(dst2d.at[b], dst_v[s], dl[s])
            if legather:
                pltpu.async_copy(eid2d.at[b], eid_v[s], si[s])
            else:
                pltpu.async_copy(les[p].at[pl.ds(b * BE, BE)], le_v[s],
                                 dl[s])

        def compute(s):
            def edge(e, _):
                for j in range(4):
                    m = (
                        jnp.maximum(
                            rows_v[s][e, pl.ds(co + 16 * j, 16)]
                            + le_v[s][e, pl.ds(cole + 16 * j, 16)], 0.0)
                        + 1e-7
                    )
                    ex = jnp.exp(m)
                    upd_v[e, pl.ds(16 * j, 16)] = m * ex
                    upd_v[e, pl.ds(64 + 16 * j, 16)] = ex
                return 0

            lax.fori_loop(0, BE, edge, 0)

        issue_idx(0, 0)
        issue_idx(1, 1)

        def pair(q, _):
            t0 = 2 * q
            for s in range(2):
                pltpu.make_async_copy(src2d.at[wid], src_v[s], si[s]).wait()
                if legather:
                    pltpu.make_async_copy(src2d.at[wid], eid_v[s],
                                          si[s]).wait()
                    pltpu.async_copy(les[0].at[eid_v[s]], le_v[s], gl[s])
                pltpu.async_copy(xt.at[src_v[s]], rows_v[s], gs[s])
            for s in range(2):
                b = wid + NW * (t0 + s)
                pltpu.make_async_copy(xt.at[src_v[s]], rows_v[s],
                                      gs[s]).wait()
                pltpu.make_async_copy(dst2d.at[wid], dst_v[s], dl[s]).wait()
                if legather:
                    pltpu.make_async_copy(les[0].at[eid_v[s]], le_v[s],
                                          gl[s]).wait()
                else:
                    pltpu.make_async_copy(les[p].at[pl.ds(b * BE, BE)],
                                          le_v[s], dl[s]).wait()
                compute(s)
                pltpu.async_copy(upd_v, acc.at[dst_v[s]], ss, add=True)
                pltpu.make_async_copy(upd_v, acc.at[dst_v[s]], ss).wait()

                @pl.when(t0 + s + 2 < T)
                def _():
                    issue_idx(t0 + s + 2, s)

            return 0

        lax.fori_loop(0, T // 2, pair, 0)
        plsc.subcore_barrier()
        for t in range(rpt // BE):
            r0 = sid * rpt + t * BE
            pltpu.sync_copy(acc.at[pl.ds(r0, BE)],
                            outs[p].at[cid, pl.ds(r0, BE)])
        plsc.subcore_barrier()


def _agg_call(xts, les, coffs, src2d, dst2d, n_acc, eid2d=None, cole=0):
    P = len(coffs)
    NT = len(xts)
    nb = src2d.shape[0]
    legather = eid2d is not None
    lef = 128 if legather else F
    mesh = plsc.VectorSubcoreMesh(core_axis_name="c", subcore_axis_name="s")
    args = [*xts, *les, src2d, dst2d] + ([eid2d] if legather else [])
    return pl.kernel(
        functools.partial(_agg_body, P, n_acc, NT, coffs, nb, legather,
                          cole),
        mesh=mesh,
        out_type=[jax.ShapeDtypeStruct((2, n_acc, 128), jnp.float32)
                  for _ in range(P)],
        scratch_types=[
            pltpu.VMEM_SHARED((n_acc, 128), jnp.float32),
            pltpu.VMEM((BE,), jnp.int32),
            pltpu.VMEM((BE,), jnp.int32),
            pltpu.VMEM((BE,), jnp.int32),
            pltpu.VMEM((BE,), jnp.int32),
            pltpu.VMEM((BE,), jnp.int32),
            pltpu.VMEM((BE,), jnp.int32),
            pltpu.VMEM((BE, 128), jnp.float32),
            pltpu.VMEM((BE, 128), jnp.float32),
            pltpu.VMEM((BE, lef), jnp.float32),
            pltpu.VMEM((BE, lef), jnp.float32),
            pltpu.VMEM((BE, 2 * F), jnp.float32),
        ] + [pltpu.SemaphoreType.DMA] * 9,
    )(*args)


def _mlpa_body(P, n, x_ref, w1_ref, b1_ref, *refs):
    accs = refs[0:P]
    h1_ref, st_ref = refs[P], refs[P + 1]
    i = pl.program_id(0)
    parts = []
    for p in range(P):
        num = accs[p][0, :, 0:F] + accs[p][1, :, 0:F]
        den = accs[p][0, :, F:2 * F] + accs[p][1, :, F:2 * F]
        parts.append(num / (den + 1e-16) + x_ref[:, p * F:(p + 1) * F])
    agg = jnp.concatenate(parts, axis=1)
    h = jnp.dot(agg, w1_ref[...], preferred_element_type=jnp.float32) \
        + b1_ref[...]
    h1_ref[...] = h

    @pl.when(i == 0)
    def _():
        st_ref[...] = jnp.zeros_like(st_ref)

    st_ref[0:1, :] += jnp.sum(h, axis=0, keepdims=True)
    st_ref[1:2, :] += jnp.sum(h * h, axis=0, keepdims=True)


def _mlpa_call(accs, x, w1, b1, n, BR):
    P = len(accs)
    Fh = w1.shape[1]
    return pl.pallas_call(
        functools.partial(_mlpa_body, P, n),
        grid=(n // BR,),
        in_specs=[
            pl.BlockSpec((BR, P * F), lambda i: (i, 0)),
            pl.BlockSpec(w1.shape, lambda i: (0, 0)),
            pl.BlockSpec((1, Fh), lambda i: (0, 0)),
        ] + [pl.BlockSpec((2, BR, 2 * F), lambda i: (0, i, 0))
             for _ in range(P)],
        out_specs=[
            pl.BlockSpec((BR, Fh), lambda i: (i, 0)),
            pl.BlockSpec((2, Fh), lambda i: (0, 0)),
        ],
        out_shape=[
            jax.ShapeDtypeStruct((n, Fh), jnp.float32),
            jax.ShapeDtypeStruct((2, Fh), jnp.float32),
        ],
    )(x, w1, b1.reshape(1, -1), *accs)


def _mlpb_body(n, h1_ref, st_ref, g_ref, be_ref, w2_ref, b2_ref, pw_ref,
               h2_ref, s_ref):
    mu = st_ref[0:1, :] / n
    var = st_ref[1:2, :] / n - mu * mu
    hn = (h1_ref[...] - mu) * jax.lax.rsqrt(var + 1e-5) * g_ref[...] \
        + be_ref[...]
    hn = jnp.maximum(hn, 0.0)
    h2 = jnp.dot(hn, w2_ref[...], preferred_element_type=jnp.float32) \
        + b2_ref[...]
    h2_ref[...] = h2
    pw = pw_ref[...]
    nrm = jnp.sqrt(jnp.sum(pw * pw)) + 1e-16
    s_ref[...] = jnp.tanh(jnp.sum(h2 * pw, axis=1, keepdims=True) / nrm)


def _mlpb_call(h1, st, g, be, w2, b2, pw, n, BR):
    Fh, Fo = w2.shape
    return pl.pallas_call(
        functools.partial(_mlpb_body, n),
        grid=(n // BR,),
        in_specs=[
            pl.BlockSpec((BR, Fh), lambda i: (i, 0)),
            pl.BlockSpec((2, Fh), lambda i: (0, 0)),
            pl.BlockSpec((1, Fh), lambda i: (0, 0)),
            pl.BlockSpec((1, Fh), lambda i: (0, 0)),
            pl.BlockSpec((Fh, Fo), lambda i: (0, 0)),
            pl.BlockSpec((1, Fo), lambda i: (0, 0)),
            pl.BlockSpec((1, Fo), lambda i: (0, 0)),
        ],
        out_specs=[
            pl.BlockSpec((BR, Fo), lambda i: (i, 0)),
            pl.BlockSpec((BR, 1), lambda i: (i, 0)),
        ],
        out_shape=[
            jax.ShapeDtypeStruct((n, Fo), jnp.float32),
            jax.ShapeDtypeStruct((n, 1), jnp.float32),
        ],
    )(h1, st, g.reshape(1, -1), be.reshape(1, -1), w2, b2.reshape(1, -1),
      pw.reshape(1, -1))


def _rank_body(k, n, ncp, CJ, BR, Fo, sc_ref, srow_ref, h2_ref, nm_ref,
               g_ref):
    i = pl.program_id(0)
    s_i = sc_ref[...]
    g = jnp.maximum(h2_ref[...] * s_i, 0.0)
    if Fo == 128:
        g_ref[...] = g
    else:
        g_ref[...] = jnp.concatenate(
            [g, jnp.zeros((BR, 128 - Fo), jnp.float32)], axis=1)
    row_id = i * BR + lax.broadcasted_iota(jnp.int32, (BR, CJ), 0)

    def chunk(j, rank):
        s_j = srow_ref[:, pl.ds(j * CJ, CJ)]
        col_id = j * CJ + lax.broadcasted_iota(jnp.int32, (BR, CJ), 1)
        gt = s_j > s_i
        eq_lt = (s_j == s_i) & (col_id < row_id)
        return rank + jnp.sum((gt | eq_lt).astype(jnp.int32), axis=1,
                              keepdims=True)

    rank = lax.fori_loop(0, ncp, chunk, jnp.zeros((BR, 1), jnp.int32))
    nm_ref[...] = jnp.where(rank < k, rank, -1)


def _rank_call(score, srow_pad, h2, k, n, BR, CJ):
    ncp = srow_pad.shape[1] // CJ
    Fo = h2.shape[1]
    return pl.pallas_call(
        functools.partial(_rank_body, k, n, ncp, CJ, BR, Fo),
        grid=(n // BR,),
        in_specs=[
            pl.BlockSpec((BR, 1), lambda i: (i, 0)),
            pl.BlockSpec(srow_pad.shape, lambda i: (0, 0)),
            pl.BlockSpec((BR, Fo), lambda i: (i, 0)),
        ],
        out_specs=[
            pl.BlockSpec((BR, 1), lambda i: (i, 0)),
            pl.BlockSpec((BR, 128), lambda i: (i, 0)),
        ],
        out_shape=[
            jax.ShapeDtypeStruct((n, 1), jnp.int32),
            jax.ShapeDtypeStruct((n, 128), jnp.float32),
        ],
    )(score, srow_pad, h2)


def _compact_body(k, n_pad, CC, spread, g_hbm, nm_hbm, out_hbm,
                  g_v, nm_v, idx_v, sem):
    cid = lax.axis_index("c")
    sid = lax.axis_index("s")
    wid = sid * 2 + cid
    npw = n_pad // NW
    lanes = lax.iota(jnp.int32, 16)
    for c in range(npw // CC):
        r0 = wid * npw + c * CC
        pltpu.sync_copy(g_hbm.at[pl.ds(r0, CC)], g_v)
        pltpu.sync_copy(nm_hbm.at[pl.ds(r0, CC)], nm_v)

        def mk(t, _):
            nid = nm_v[pl.ds(t * 16, 16)]
            glob = r0 + lanes + t * 16
            dump = k + (glob & (spread - 1))
            idx_v[pl.ds(t * 16, 16)] = jnp.where(nid >= 0, nid, dump)
            return 0

        lax.fori_loop(0, CC // 16, mk, 0)
        pltpu.async_copy(g_v, out_hbm.at[idx_v], sem).wait()


def _compact_call(g_pad, nm_pad, k, CC, spread):
    n_pad = g_pad.shape[0]
    mesh = plsc.VectorSubcoreMesh(core_axis_name="c", subcore_axis_name="s")
    return pl.kernel(
        functools.partial(_compact_body, k, n_pad, CC, spread),
        mesh=mesh,
        out_type=jax.ShapeDtypeStruct((k + spread, 128), jnp.float32),
        scratch_types=[
            pltpu.VMEM((CC, 128), jnp.float32),
            pltpu.VMEM((CC,), jnp.int32),
            pltpu.VMEM((CC,), jnp.int32),
            pltpu.SemaphoreType.DMA,
        ],
    )(g_pad, nm_pad)


def _remap_body(k_new, smask, nb, src2d, dst2d, nm_hbm, souts, douts,
                si_v, di_v, ns_v, nd_v, so_v, do_v, sem):
    cid = lax.axis_index("c")
    sid = lax.axis_index("s")
    wid = sid * 2 + cid
    lanes = lax.iota(jnp.int32, 16)

    def blk(t, _):
        b = wid + NW * t
        pltpu.sync_copy(src2d.at[b], si_v)
        pltpu.sync_copy(dst2d.at[b], di_v)
        pltpu.async_copy(nm_hbm.at[si_v], ns_v, sem).wait()
        pltpu.async_copy(nm_hbm.at[di_v], nd_v, sem).wait()

        def grp(t2, _):
            sl = pl.ds(t2 * 16, 16)
            ns = ns_v[sl]
            nd = nd_v[sl]
            ok = (ns >= 0) & (nd >= 0)
            ge = b * BE + lanes + t2 * 16
            so_v[sl] = jnp.where(ok, ns, ge & (smask - 1))
            do_v[sl] = jnp.where(ok, nd, k_new + (ge & (DUMP - 1)))
            return 0

        lax.fori_loop(0, BE // 16, grp, 0)
        pltpu.sync_copy(so_v, souts.at[b])
        pltpu.sync_copy(do_v, douts.at[b])
        return 0

    lax.fori_loop(0, nb // NW, blk, 0)


def _remap_call(src2d, dst2d, nm_pad, k_new, smask):
    nb = src2d.shape[0]
    mesh = plsc.VectorSubcoreMesh(core_axis_name="c", subcore_axis_name="s")
    return pl.kernel(
        functools.partial(_remap_body, k_new, smask, nb),
        mesh=mesh,
        out_type=[jax.ShapeDtypeStruct((nb, BE), jnp.int32),
                  jax.ShapeDtypeStruct((nb, BE), jnp.int32)],
        scratch_types=[
            pltpu.VMEM((BE,), jnp.int32),
            pltpu.VMEM((BE,), jnp.int32),
            pltpu.VMEM((BE,), jnp.int32),
            pltpu.VMEM((BE,), jnp.int32),
            pltpu.VMEM((BE,), jnp.int32),
            pltpu.VMEM((BE,), jnp.int32),
            pltpu.SemaphoreType.DMA,
        ],
    )(src2d, dst2d, nm_pad)


def _head_body(x3_ref, d1w_ref, d1b_ref, d2w_ref, d2b_ref, o_ref):
    gm = jnp.sum(x3_ref[...], axis=0, keepdims=True) / K3
    h = jnp.dot(gm, d1w_ref[...], preferred_element_type=jnp.float32) \
        + d1b_ref[...]
    o = jnp.dot(h, d2w_ref[...], preferred_element_type=jnp.float32) \
        + d2b_ref[...]
    m = jnp.max(o, axis=-1, keepdims=True)
    z = o - m
    o_ref[...] = z - jnp.log(jnp.sum(jnp.exp(z), axis=-1, keepdims=True))


def _head_call(x3, d1_w, d1_b, d2_w, d2_b):
    return pl.pallas_call(
        _head_body,
        out_shape=jax.ShapeDtypeStruct((1, 10), jnp.float32),
    )(x3[:K3], d1_w, d1_b.reshape(1, -1), d2_w, d2_b.reshape(1, -1))


def _pad_rows(a, rows, value=0.0):
    return jnp.pad(a, ((0, rows - a.shape[0]), (0, 0)),
                   constant_values=value)


def _pad1d(a, size, value):
    return jnp.pad(a, (0, size - a.shape[0]), constant_values=value)


def kernel(x, edge_index, edge_attr, batch, c1_lew, c1_leb, c1_w1, c1_b1, c1_g, c1_be, c1_w2, c1_b2, c2_lew, c2_leb, c2_w1, c2_b1, c2_g, c2_be, c2_w2, c2_b2, c3_lew, c3_leb, c3_w1, c3_b1, c3_g, c3_be, c3_w2, c3_b2, p1_w, p2_w, p3_w, d1_w, d1_b, d2_w, d2_b):
    ap = jnp.arange(E, E_PAD, dtype=jnp.int32)
    src_pad = jnp.concatenate([edge_index[0], ap & 8191])
    dst_pad = jnp.concatenate([edge_index[1], N + (ap % 240)])
    src2d = src_pad.reshape(NBP, BE)
    dst2d = dst_pad.reshape(NBP, BE)
    ea_pad = _pad_rows(edge_attr, E_PAD)

    le4 = _le_call(ea_pad, c1_lew, c1_leb, 4)
    xt = x.reshape(N, 2, 128).transpose(1, 0, 2)
    accs = _agg_call([xt[0], xt[1]], [le4[p] for p in range(4)],
                     [0, 64, 0, 64], src2d, dst2d, 10240)
    h1, st = _mlpa_call(accs, x, c1_w1, c1_b1, N, 400)
    h2, score = _mlpb_call(h1, st, c1_g, c1_be, c1_w2, c1_b2, p1_w, N, 400)
    srow = _pad_rows(score, 10240, -1e30).reshape(1, 10240)
    nm1, g1 = _rank_call(score, srow, h2, K1, N, 400, 1024)
    nm1d = nm1.reshape(-1)
    x2 = _compact_call(_pad_rows(g1, 10240), _pad1d(nm1d, 10240, -1),
                       K1, 64, 1024)
    src2, dst2 = _remap_call(src2d, dst2d, _pad1d(nm1d, 10240, -1),
                             K1, 1024)

    le2 = _le_call(ea_pad, c2_lew, c2_leb, 1)
    accs2 = _agg_call([x2], [le2[0]], [0], src2, dst2, 4096)
    h1, st = _mlpa_call(accs2, x2[:K1, :F], c2_w1, c2_b1, K1, 400)
    h2, score = _mlpb_call(h1, st, c2_g, c2_be, c2_w2, c2_b2, p2_w, K1, 400)
    srow = _pad_rows(score, 2048, -1e30).reshape(1, 2048)
    nm2, g2 = _rank_call(score, srow, h2, K2, K1, 400, 1024)
    nm2d = nm2.reshape(-1)
    x3 = _compact_call(_pad_rows(g2, 2048), _pad1d(nm2d, 2048, -1),
                       K2, 64, 1024)
    src3, dst3 = _remap_call(src2, dst2, _pad1d(nm2d, 4096, -1), K2, 256)

    le3 = _le_call(ea_pad, c3_lew, c3_leb, 1)
    accs3 = _agg_call([x3], [le3[0]], [0], src3, dst3, 4096)
    h1, st = _mlpa_call(accs3, x3[:K2, :F], c3_w1, c3_b1, K2, 400)
    h2, score = _mlpb_call(h1, st, c3_g, c3_be, c3_w2, c3_b2, p3_w, K2, 400)
    srow = _pad_rows(score, 512, -1e30).reshape(1, 512)
    nm3, g3 = _rank_call(score, srow, h2, K3, K2, 400, 512)
    x4f = _compact_call(_pad_rows(g3, 512), _pad1d(nm3.reshape(-1), 512, -1),
                        K3, 16, 1024)

    return _head_call(x4f, d1_w, d1_b, d2_w, d2_b)


# speedup vs baseline: 1.0309x; 1.0309x over previous
import functools

import jax
import jax.numpy as jnp
from jax import lax
from jax.experimental import pallas as pl
from jax.experimental.pallas import tpu as pltpu
from jax.experimental.pallas import tpu_sc as plsc

N, E, D, ED = 10000, 160000, 256, 16
K1, K2, K3 = 2000, 400, 80
BE = 64
NB = E // BE
NW = 32
TB = (NB + NW - 1) // NW
E_PAD = 163840
NBP = E_PAD // BE
DUMP = 2048
F = 64


def _le_body(ea_ref, lew_ref, leb_ref, out_ref):
    out_ref[...] = (
        jnp.dot(ea_ref[...], lew_ref[...], preferred_element_type=jnp.float32)
        + leb_ref[...]
    )


def _le_call(ea, lew, leb, P):
    e_pad = ea.shape[0]
    lewp = lew.reshape(ED, P, F).transpose(1, 0, 2)
    lebp = leb.reshape(P, 1, F)
    return pl.pallas_call(
        _le_body,
        grid=(e_pad // 640, P),
        in_specs=[
            pl.BlockSpec((640, ED), lambda i, p: (i, 0)),
            pl.BlockSpec((None, ED, F), lambda i, p: (p, 0, 0)),
            pl.BlockSpec((None, 1, F), lambda i, p: (p, 0, 0)),
        ],
        out_specs=pl.BlockSpec((None, 640, F), lambda i, p: (p, i, 0)),
        out_shape=jax.ShapeDtypeStruct((P, e_pad, F), jnp.float32),
    )(ea, lewp, lebp)


def _le23_body(ea_ref, w_ref, b_ref, out_ref):
    out_ref[...] = (
        jnp.dot(ea_ref[...], w_ref[...], preferred_element_type=jnp.float32)
        + b_ref[...]
    )


def _le23_call(ea, lew2, leb2, lew3, leb3):
    e_pad = ea.shape[0]
    w23 = jnp.concatenate([lew2, lew3], axis=1)
    b23 = jnp.concatenate([leb2, leb3]).reshape(1, 128)
    return pl.pallas_call(
        _le23_body,
        grid=(e_pad // 640,),
        in_specs=[
            pl.BlockSpec((640, ED), lambda i: (i, 0)),
            pl.BlockSpec((ED, 128), lambda i: (0, 0)),
            pl.BlockSpec((1, 128), lambda i: (0, 0)),
        ],
        out_specs=pl.BlockSpec((640, 128), lambda i: (i, 0)),
        out_shape=jax.ShapeDtypeStruct((e_pad, 128), jnp.float32),
    )(ea, w23, b23)


def _agg_body(P, n_acc, NT, coffs, nb, legather, cole, *refs):
    xts = refs[0:NT]
    nles = 1 if legather else P
    les = refs[NT:NT + nles]
    src2d, dst2d = refs[NT + nles], refs[NT + nles + 1]
    i = NT + nles + 2
    if legather:
        eid2d = refs[i]
        i += 1
    outs = refs[i:i + P]
    (acc, src_v0, src_v1, dst_v0, dst_v1, eid_v0, eid_v1, rows_v0, rows_v1,
     le_v0, le_v1, upd_v, si0, si1, dl0, dl1, gs0, gs1, gl0, gl1,
     ss) = refs[i + P:]
    src_v = (src_v0, src_v1)
    dst_v = (dst_v0, dst_v1)
    eid_v = (eid_v0, eid_v1)
    rows_v = (rows_v0, rows_v1)
    le_v = (le_v0, le_v1)
    si = (si0, si1)
    dl = (dl0, dl1)
    gs = (gs0, gs1)
    gl = (gl0, gl1)
    T = nb // NW

    cid = lax.axis_index("c")
    sid = lax.axis_index("s")
    wid = sid * 2 + cid
    rpt = n_acc // 16

    for p in range(P):
        xt = xts[p // (P // NT)]
        co = coffs[p]

        def zero_upd(r, _):
            for j in range(8):
                upd_v[r, pl.ds(16 * j, 16)] = jnp.zeros((16,), jnp.float32)
            return 0

        lax.fori_loop(0, BE, zero_upd, 0)
        for t in range(rpt // BE):
            pltpu.sync_copy(upd_v, acc.at[pl.ds(sid * rpt + t * BE, BE)])
        plsc.subcore_barrier()

        def issue_idx(t, s):
            b = wid + NW * t
            pltpu.async_copy(src2d.at[b], src_v[s], si[s])
            pltpu.async_copy(dst2d.at[b], dst_v[s], dl[s])
            if legather:
                pltpu.async_copy(eid2d.at[b], eid_v[s], si[s])
            else:
                pltpu.async_copy(les[p].at[pl.ds(b * BE, BE)], le_v[s],
                                 dl[s])

        def compute(s):
            def edge(e, _):
                for j in range(4):
                    m = (
                        jnp.maximum(
                            rows_v[s][e, pl.ds(co + 16 * j, 16)]
                            + le_v[s][e, pl.ds(cole + 16 * j, 16)], 0.0)
                        + 1e-7
                    )
                    ex = jnp.exp(m)
                    upd_v[e, pl.ds(16 * j, 16)] = m * ex
                    upd_v[e, pl.ds(64 + 16 * j, 16)] = ex
                return 0

            lax.fori_loop(0, BE, edge, 0)

        issue_idx(0, 0)
        issue_idx(1, 1)

        def pair(q, _):
            t0 = 2 * q
            for s in range(2):
                pltpu.make_async_copy(src2d.at[wid], src_v[s], si[s]).wait()
                if legather:
                    pltpu.make_async_copy(src2d.at[wid], eid_v[s],
                                          si[s]).wait()
                    pltpu.async_copy(les[0].at[eid_v[s]], le_v[s], gl[s])
                pltpu.async_copy(xt.at[src_v[s]], rows_v[s], gs[s])
            for s in range(2):
                b = wid + NW * (t0 + s)
                pltpu.make_async_copy(xt.at[src_v[s]], rows_v[s],
                                      gs[s]).wait()
                pltpu.make_async_copy(dst2d.at[wid], dst_v[s], dl[s]).wait()
                if legather:
                    pltpu.make_async_copy(les[0].at[eid_v[s]], le_v[s],
                                          gl[s]).wait()
                else:
                    pltpu.make_async_copy(les[p].at[pl.ds(b * BE, BE)],
                                          le_v[s], dl[s]).wait()
                compute(s)
                pltpu.async_copy(upd_v, acc.at[dst_v[s]], ss, add=True)
                pltpu.make_async_copy(upd_v, acc.at[dst_v[s]], ss).wait()

                @pl.when(t0 + s + 2 < T)
                def _():
                    issue_idx(t0 + s + 2, s)

            return 0

        lax.fori_loop(0, T // 2, pair, 0)
        plsc.subcore_barrier()
        for t in range(rpt // BE):
            r0 = sid * rpt + t * BE
            pltpu.sync_copy(acc.at[pl.ds(r0, BE)],
                            outs[p].at[cid, pl.ds(r0, BE)])
        plsc.subcore_barrier()


def _agg_call(xts, les, coffs, src2d, dst2d, n_acc, eid2d=None, cole=0):
    P = len(coffs)
    NT = len(xts)
    nb = src2d.shape[0]
    legather = eid2d is not None
    lef = 128 if legather else F
    mesh = plsc.VectorSubcoreMesh(core_axis_name="c", subcore_axis_name="s")
    args = [*xts, *les, src2d, dst2d] + ([eid2d] if legather else [])
    return pl.kernel(
        functools.partial(_agg_body, P, n_acc, NT, coffs, nb, legather,
                          cole),
        mesh=mesh,
        out_type=[jax.ShapeDtypeStruct((2, n_acc, 128), jnp.float32)
                  for _ in range(P)],
        scratch_types=[
            pltpu.VMEM_SHARED((n_acc, 128), jnp.float32),
            pltpu.VMEM((BE,), jnp.int32),
            pltpu.VMEM((BE,), jnp.int32),
            pltpu.VMEM((BE,), jnp.int32),
            pltpu.VMEM((BE,), jnp.int32),
            pltpu.VMEM((BE,), jnp.int32),
            pltpu.VMEM((BE,), jnp.int32),
            pltpu.VMEM((BE, 128), jnp.float32),
            pltpu.VMEM((BE, 128), jnp.float32),
            pltpu.VMEM((BE, lef), jnp.float32),
            pltpu.VMEM((BE, lef), jnp.float32),
            pltpu.VMEM((BE, 2 * F), jnp.float32),
        ] + [pltpu.SemaphoreType.DMA] * 9,
    )(*args)


def _mlpa_body(P, n, x_ref, w1_ref, b1_ref, *refs):
    accs = refs[0:P]
    h1_ref, st_ref = refs[P], refs[P + 1]
    i = pl.program_id(0)
    parts = []
    for p in range(P):
        num = accs[p][0, :, 0:F] + accs[p][1, :, 0:F]
        den = accs[p][0, :, F:2 * F] + accs[p][1, :, F:2 * F]
        parts.append(num / (den + 1e-16) + x_ref[:, p * F:(p + 1) * F])
    agg = jnp.concatenate(parts, axis=1)
    h = jnp.dot(agg, w1_ref[...], preferred_element_type=jnp.float32) \
        + b1_ref[...]
    h1_ref[...] = h

    @pl.when(i == 0)
    def _():
        st_ref[...] = jnp.zeros_like(st_ref)

    st_ref[0:1, :] += jnp.sum(h, axis=0, keepdims=True)
    st_ref[1:2, :] += jnp.sum(h * h, axis=0, keepdims=True)


def _mlpa_call(accs, x, w1, b1, n, BR):
    P = len(accs)
    Fh = w1.shape[1]
    return pl.pallas_call(
        functools.partial(_mlpa_body, P, n),
        grid=(n // BR,),
        in_specs=[
            pl.BlockSpec((BR, P * F), lambda i: (i, 0)),
            pl.BlockSpec(w1.shape, lambda i: (0, 0)),
            pl.BlockSpec((1, Fh), lambda i: (0, 0)),
        ] + [pl.BlockSpec((2, BR, 2 * F), lambda i: (0, i, 0))
             for _ in range(P)],
        out_specs=[
            pl.BlockSpec((BR, Fh), lambda i: (i, 0)),
            pl.BlockSpec((2, Fh), lambda i: (0, 0)),
        ],
        out_shape=[
            jax.ShapeDtypeStruct((n, Fh), jnp.float32),
            jax.ShapeDtypeStruct((2, Fh), jnp.float32),
        ],
    )(x, w1, b1.reshape(1, -1), *accs)


def _mlpb_body(n, h1_ref, st_ref, g_ref, be_ref, w2_ref, b2_ref, pw_ref,
               h2_ref, s_ref):
    mu = st_ref[0:1, :] / n
    var = st_ref[1:2, :] / n - mu * mu
    hn = (h1_ref[...] - mu) * jax.lax.rsqrt(var + 1e-5) * g_ref[...] \
        + be_ref[...]
    hn = jnp.maximum(hn, 0.0)
    h2 = jnp.dot(hn, w2_ref[...], preferred_element_type=jnp.float32) \
        + b2_ref[...]
    h2_ref[...] = h2
    pw = pw_ref[...]
    nrm = jnp.sqrt(jnp.sum(pw * pw)) + 1e-16
    s_ref[...] = jnp.tanh(jnp.sum(h2 * pw, axis=1, keepdims=True) / nrm)


def _mlpb_call(h1, st, g, be, w2, b2, pw, n, BR):
    Fh, Fo = w2.shape
    return pl.pallas_call(
        functools.partial(_mlpb_body, n),
        grid=(n // BR,),
        in_specs=[
            pl.BlockSpec((BR, Fh), lambda i: (i, 0)),
            pl.BlockSpec((2, Fh), lambda i: (0, 0)),
            pl.BlockSpec((1, Fh), lambda i: (0, 0)),
            pl.BlockSpec((1, Fh), lambda i: (0, 0)),
            pl.BlockSpec((Fh, Fo), lambda i: (0, 0)),
            pl.BlockSpec((1, Fo), lambda i: (0, 0)),
            pl.BlockSpec((1, Fo), lambda i: (0, 0)),
        ],
        out_specs=[
            pl.BlockSpec((BR, Fo), lambda i: (i, 0)),
            pl.BlockSpec((BR, 1), lambda i: (i, 0)),
        ],
        out_shape=[
            jax.ShapeDtypeStruct((n, Fo), jnp.float32),
            jax.ShapeDtypeStruct((n, 1), jnp.float32),
        ],
    )(h1, st, g.reshape(1, -1), be.reshape(1, -1), w2, b2.reshape(1, -1),
      pw.reshape(1, -1))


def _rank_body(k, n, ncp, CJ, BR, Fo, sc_ref, srow_ref, h2_ref, nm_ref,
               g_ref):
    i = pl.program_id(0)
    s_i = sc_ref[...]
    g = jnp.maximum(h2_ref[...] * s_i, 0.0)
    if Fo == 128:
        g_ref[...] = g
    else:
        g_ref[...] = jnp.concatenate(
            [g, jnp.zeros((BR, 128 - Fo), jnp.float32)], axis=1)
    row_id = i * BR + lax.broadcasted_iota(jnp.int32, (BR, CJ), 0)

    def chunk(j, rank):
        s_j = srow_ref[:, pl.ds(j * CJ, CJ)]
        col_id = j * CJ + lax.broadcasted_iota(jnp.int32, (BR, CJ), 1)
        gt = s_j > s_i
        eq_lt = (s_j == s_i) & (col_id < row_id)
        return rank + jnp.sum((gt | eq_lt).astype(jnp.int32), axis=1,
                              keepdims=True)

    rank = lax.fori_loop(0, ncp, chunk, jnp.zeros((BR, 1), jnp.int32))
    nm_ref[...] = jnp.where(rank < k, rank, -1)


def _rank_call(score, srow_pad, h2, k, n, BR, CJ):
    ncp = srow_pad.shape[1] // CJ
    Fo = h2.shape[1]
    return pl.pallas_call(
        functools.partial(_rank_body, k, n, ncp, CJ, BR, Fo),
        grid=(n // BR,),
        in_specs=[
            pl.BlockSpec((BR, 1), lambda i: (i, 0)),
            pl.BlockSpec(srow_pad.shape, lambda i: (0, 0)),
            pl.BlockSpec((BR, Fo), lambda i: (i, 0)),
        ],
        out_specs=[
            pl.BlockSpec((BR, 1), lambda i: (i, 0)),
            pl.BlockSpec((BR, 128), lambda i: (i, 0)),
        ],
        out_shape=[
            jax.ShapeDtypeStruct((n, 1), jnp.int32),
            jax.ShapeDtypeStruct((n, 128), jnp.float32),
        ],
    )(score, srow_pad, h2)


def _compact_body(k, n_pad, CC, spread, g_hbm, nm_hbm, out_hbm,
                  g_v, nm_v, idx_v, sem):
    cid = lax.axis_index("c")
    sid = lax.axis_index("s")
    wid = sid * 2 + cid
    npw = n_pad // NW
    lanes = lax.iota(jnp.int32, 16)
    for c in range(npw // CC):
        r0 = wid * npw + c * CC
        pltpu.sync_copy(g_hbm.at[pl.ds(r0, CC)], g_v)
        pltpu.sync_copy(nm_hbm.at[pl.ds(r0, CC)], nm_v)

        def mk(t, _):
            nid = nm_v[pl.ds(t * 16, 16)]
            glob = r0 + lanes + t * 16
            dump = k + (glob & (spread - 1))
            idx_v[pl.ds(t * 16, 16)] = jnp.where(nid >= 0, nid, dump)
            return 0

        lax.fori_loop(0, CC // 16, mk, 0)
        pltpu.async_copy(g_v, out_hbm.at[idx_v], sem).wait()


def _compact_call(g_pad, nm_pad, k, CC, spread):
    n_pad = g_pad.shape[0]
    mesh = plsc.VectorSubcoreMesh(core_axis_name="c", subcore_axis_name="s")
    return pl.kernel(
        functools.partial(_compact_body, k, n_pad, CC, spread),
        mesh=mesh,
        out_type=jax.ShapeDtypeStruct((k + spread, 128), jnp.float32),
        scratch_types=[
            pltpu.VMEM((CC, 128), jnp.float32),
            pltpu.VMEM((CC,), jnp.int32),
            pltpu.VMEM((CC,), jnp.int32),
            pltpu.SemaphoreType.DMA,
        ],
    )(g_pad, nm_pad)


def _remap_body(k_new, smask, nb, *refs):
    (src2d, dst2d, nm_hbm, souts, douts,
     si_v0, si_v1, di_v0, di_v1, ns_v0, ns_v1, nd_v0, nd_v1,
     so_v0, so_v1, do_v0, do_v1,
     li0, li1, gn0, gn1, ws0, ws1) = refs
    si_v = (si_v0, si_v1)
    di_v = (di_v0, di_v1)
    ns_v = (ns_v0, ns_v1)
    nd_v = (nd_v0, nd_v1)
    so_v = (so_v0, so_v1)
    do_v = (do_v0, do_v1)
    li = (li0, li1)
    gn = (gn0, gn1)
    ws = (ws0, ws1)
    T = nb // NW
    cid = lax.axis_index("c")
    sid = lax.axis_index("s")
    wid = sid * 2 + cid
    lanes = lax.iota(jnp.int32, 16)

    def issue_idx(t, s):
        b = wid + NW * t
        pltpu.async_copy(src2d.at[b], si_v[s], li[s])
        pltpu.async_copy(dst2d.at[b], di_v[s], li[s])

    issue_idx(0, 0)
    issue_idx(1, 1)

    def pair(q, _):
        t0 = 2 * q
        for s in range(2):
            pltpu.make_async_copy(src2d.at[wid], si_v[s], li[s]).wait()
            pltpu.make_async_copy(src2d.at[wid], di_v[s], li[s]).wait()
            pltpu.async_copy(nm_hbm.at[si_v[s]], ns_v[s], gn[s])
            pltpu.async_copy(nm_hbm.at[di_v[s]], nd_v[s], gn[s])
        for s in range(2):
            b = wid + NW * (t0 + s)
            pltpu.make_async_copy(nm_hbm.at[si_v[s]], ns_v[s], gn[s]).wait()
            pltpu.make_async_copy(nm_hbm.at[di_v[s]], nd_v[s], gn[s]).wait()

            @pl.when(q > 0)
            def _():
                pltpu.make_async_copy(so_v[s], souts.at[wid], ws[s]).wait()
                pltpu.make_async_copy(do_v[s], douts.at[wid], ws[s]).wait()

            def grp(t2, _):
                sl = pl.ds(t2 * 16, 16)
                ns = ns_v[s][sl]
                nd = nd_v[s][sl]
                ok = (ns >= 0) & (nd >= 0)
                ge = b * BE + lanes + t2 * 16
                so_v[s][sl] = jnp.where(ok, ns, ge & (smask - 1))
                do_v[s][sl] = jnp.where(ok, nd, k_new + (ge & (DUMP - 1)))
                return 0

            lax.fori_loop(0, BE // 16, grp, 0)
            pltpu.async_copy(so_v[s], souts.at[b], ws[s])
            pltpu.async_copy(do_v[s], douts.at[b], ws[s])

            @pl.when(t0 + s + 2 < T)
            def _():
                issue_idx(t0 + s + 2, s)

        return 0

    lax.fori_loop(0, T // 2, pair, 0)
    for s in range(2):
        pltpu.make_async_copy(so_v[s], souts.at[wid], ws[s]).wait()
        pltpu.make_async_copy(do_v[s], douts.at[wid], ws[s]).wait()


def _remap_call(src2d, dst2d, nm_pad, k_new, smask):
    nb = src2d.shape[0]
    mesh = plsc.VectorSubcoreMesh(core_axis_name="c", subcore_axis_name="s")
    return pl.kernel(
        functools.partial(_remap_body, k_new, smask, nb),
        mesh=mesh,
        out_type=[jax.ShapeDtypeStruct((nb, BE), jnp.int32),
                  jax.ShapeDtypeStruct((nb, BE), jnp.int32)],
        scratch_types=[pltpu.VMEM((BE,), jnp.int32) for _ in range(12)]
        + [pltpu.SemaphoreType.DMA] * 6,
    )(src2d, dst2d, nm_pad)


def _head_body(x3_ref, d1w_ref, d1b_ref, d2w_ref, d2b_ref, o_ref):
    gm = jnp.sum(x3_ref[...], axis=0, keepdims=True) / K3
    h = jnp.dot(gm, d1w_ref[...], preferred_element_type=jnp.float32) \
        + d1b_ref[...]
    o = jnp.dot(h, d2w_ref[...], preferred_element_type=jnp.float32) \
        + d2b_ref[...]
    m = jnp.max(o, axis=-1, keepdims=True)
    z = o - m
    o_ref[...] = z - jnp.log(jnp.sum(jnp.exp(z), axis=-1, keepdims=True))


def _head_call(x3, d1_w, d1_b, d2_w, d2_b):
    return pl.pallas_call(
        _head_body,
        out_shape=jax.ShapeDtypeStruct((1, 10), jnp.float32),
    )(x3[:K3], d1_w, d1_b.reshape(1, -1), d2_w, d2_b.reshape(1, -1))


def _pad_rows(a, rows, value=0.0):
    return jnp.pad(a, ((0, rows - a.shape[0]), (0, 0)),
                   constant_values=value)


def _pad1d(a, size, value):
    return jnp.pad(a, (0, size - a.shape[0]), constant_values=value)


def kernel(x, edge_index, edge_attr, batch, c1_lew, c1_leb, c1_w1, c1_b1, c1_g, c1_be, c1_w2, c1_b2, c2_lew, c2_leb, c2_w1, c2_b1, c2_g, c2_be, c2_w2, c2_b2, c3_lew, c3_leb, c3_w1, c3_b1, c3_g, c3_be, c3_w2, c3_b2, p1_w, p2_w, p3_w, d1_w, d1_b, d2_w, d2_b):
    ap = jnp.arange(E, E_PAD, dtype=jnp.int32)
    src_pad = jnp.concatenate([edge_index[0], ap & 8191])
    dst_pad = jnp.concatenate([edge_index[1], N + (ap % 240)])
    src2d = src_pad.reshape(NBP, BE)
    dst2d = dst_pad.reshape(NBP, BE)
    ea_pad = _pad_rows(edge_attr, E_PAD)

    le4 = _le_call(ea_pad, c1_lew, c1_leb, 4)
    xt = x.reshape(N, 2, 128).transpose(1, 0, 2)
    accs = _agg_call([xt[0], xt[1]], [le4[p] for p in range(4)],
                     [0, 64, 0, 64], src2d, dst2d, 10240)
    h1, st = _mlpa_call(accs, x, c1_w1, c1_b1, N, 400)
    h2, score = _mlpb_call(h1, st, c1_g, c1_be, c1_w2, c1_b2, p1_w, N, 400)
    srow = _pad_rows(score, 10240, -1e30).reshape(1, 10240)
    nm1, g1 = _rank_call(score, srow, h2, K1, N, 400, 1024)
    nm1d = nm1.reshape(-1)
    x2 = _compact_call(_pad_rows(g1, 10240), _pad1d(nm1d, 10240, -1),
                       K1, 64, 1024)
    src2, dst2 = _remap_call(src2d, dst2d, _pad1d(nm1d, 10240, -1),
                             K1, 1024)

    le2 = _le_call(ea_pad, c2_lew, c2_leb, 1)
    accs2 = _agg_call([x2], [le2[0]], [0], src2, dst2, 4096)
    h1, st = _mlpa_call(accs2, x2[:K1, :F], c2_w1, c2_b1, K1, 400)
    h2, score = _mlpb_call(h1, st, c2_g, c2_be, c2_w2, c2_b2, p2_w, K1, 400)
    srow = _pad_rows(score, 2048, -1e30).reshape(1, 2048)
    nm2, g2 = _rank_call(score, srow, h2, K2, K1, 400, 1024)
    nm2d = nm2.reshape(-1)
    x3 = _compact_call(_pad_rows(g2, 2048), _pad1d(nm2d, 2048, -1),
                       K2, 64, 1024)
    src3, dst3 = _remap_call(src2, dst2, _pad1d(nm2d, 4096, -1), K2, 256)

    le3 = _le_call(ea_pad, c3_lew, c3_leb, 1)
    accs3 = _agg_call([x3], [le3[0]], [0], src3, dst3, 4096)
    h1, st = _mlpa_call(accs3, x3[:K2, :F], c3_w1, c3_b1, K2, 400)
    h2, score = _mlpb_call(h1, st, c3_g, c3_be, c3_w2, c3_b2, p3_w, K2, 400)
    srow = _pad_rows(score, 512, -1e30).reshape(1, 512)
    nm3, g3 = _rank_call(score, srow, h2, K3, K2, 400, 512)
    x4f = _compact_call(_pad_rows(g3, 512), _pad1d(nm3.reshape(-1), 512, -1),
                        K3, 16, 1024)

    return _head_call(x4f, d1_w, d1_b, d2_w, d2_b)
